# baseline (device time: 141387 ns/iter reference)
import jax
import jax.numpy as jnp
from jax import lax
from jax.experimental import pallas as pl
from jax.experimental.pallas import tpu as pltpu

N_DEV = 8
SQ = 1024
SKV = 1024
HQ_LOCAL = 8
DH = 128
D_MODEL = 1024
D_FF_LOCAL = HQ_LOCAL * DH
CHUNK = SQ // N_DEV
SCALE = 0.08838834764831843

_sem_signal = getattr(pl, "semaphore_signal", None) or pltpu.semaphore_signal
_sem_wait = getattr(pl, "semaphore_wait", None) or pltpu.semaphore_wait


def kernel(x, Wq, K_ext, V_ext, Wo):
    pos = lax.axis_index("i")
    x2 = x[0]
    k3 = K_ext[0]
    v3 = V_ext[0]
    wq_loc = lax.dynamic_slice_in_dim(Wq, pos * D_FF_LOCAL, D_FF_LOCAL, axis=1)
    wo_loc = lax.dynamic_slice_in_dim(Wo, pos * D_FF_LOCAL, D_FF_LOCAL, axis=0)

    def body(x_ref, wq_ref, k_ref, v_ref, wo_ref, out_ref,
             acc_ref, recv_ref, rs_send, rs_recv, ag_send, ag_recv):
        my = lax.axis_index("i")
        right = (my + 1) % N_DEV
        left = (my + N_DEV - 1) % N_DEV

        barrier = pltpu.get_barrier_semaphore()
        for nbr in (left, right):
            _sem_signal(barrier, inc=1, device_id=(nbr,),
                        device_id_type=pl.DeviceIdType.MESH)
        _sem_wait(barrier, 2)

        q = jnp.dot(x_ref[:, :], wq_ref[:, :],
                    preferred_element_type=jnp.float32)

        qi = lax.broadcasted_iota(jnp.int32, (SQ, SKV), 0)
        ki = lax.broadcasted_iota(jnp.int32, (SQ, SKV), 1)
        mask = (jnp.abs(qi - ki) <= 128) | (ki < 32) | (qi < 32)

        ctx_heads = []
        for h in range(HQ_LOCAL):
            q_h = q[:, h * DH:(h + 1) * DH]
            k_h = k_ref[:, h, :]
            v_h = v_ref[:, h, :]
            s = jax.lax.dot_general(
                q_h, k_h, (((1,), (1,)), ((), ())),
                preferred_element_type=jnp.float32) * SCALE
            s = jnp.where(mask, s, -1e9)
            m = jnp.max(s, axis=-1, keepdims=True)
            e = jnp.exp(s - m)
            denom = jnp.sum(e, axis=-1, keepdims=True)
            ctx_h = jnp.dot(e, v_h, preferred_element_type=jnp.float32) / denom
            ctx_heads.append(ctx_h)
        ctx = jnp.concatenate(ctx_heads, axis=1)

        acc_ref[:, :] = jnp.dot(ctx, wo_ref[:, :],
                                preferred_element_type=jnp.float32)

        for s in range(N_DEV - 1):
            c_send = (my + N_DEV - s) % N_DEV
            c_recv = (my + N_DEV - s - 1) % N_DEV
            rdma = pltpu.make_async_remote_copy(
                src_ref=acc_ref.at[pl.ds(c_send * CHUNK, CHUNK), :],
                dst_ref=recv_ref.at[s],
                send_sem=rs_send.at[s],
                recv_sem=rs_recv.at[s],
                device_id=(right,),
                device_id_type=pl.DeviceIdType.MESH,
            )
            rdma.start()
            rdma.wait_recv()
            acc_ref[pl.ds(c_recv * CHUNK, CHUNK), :] = (
                acc_ref[pl.ds(c_recv * CHUNK, CHUNK), :] + recv_ref[s]
            )
            rdma.wait_send()

        c_mine = (my + 1) % N_DEV
        out_ref[pl.ds(c_mine * CHUNK, CHUNK), :] = (
            acc_ref[pl.ds(c_mine * CHUNK, CHUNK), :]
        )

        for t in range(N_DEV - 1):
            c_send = (my + 1 + N_DEV - t) % N_DEV
            rdma = pltpu.make_async_remote_copy(
                src_ref=out_ref.at[pl.ds(c_send * CHUNK, CHUNK), :],
                dst_ref=out_ref.at[pl.ds(c_send * CHUNK, CHUNK), :],
                send_sem=ag_send.at[t],
                recv_sem=ag_recv.at[t],
                device_id=(right,),
                device_id_type=pl.DeviceIdType.MESH,
            )
            rdma.start()
            rdma.wait_recv()
            rdma.wait_send()

    out2 = pl.pallas_call(
        body,
        out_shape=jax.ShapeDtypeStruct((SQ, D_MODEL), jnp.float32),
        in_specs=[pl.BlockSpec(memory_space=pltpu.VMEM)] * 5,
        out_specs=pl.BlockSpec(memory_space=pltpu.VMEM),
        scratch_shapes=[
            pltpu.VMEM((SQ, D_MODEL), jnp.float32),
            pltpu.VMEM((N_DEV - 1, CHUNK, D_MODEL), jnp.float32),
            pltpu.SemaphoreType.DMA((N_DEV - 1,)),
            pltpu.SemaphoreType.DMA((N_DEV - 1,)),
            pltpu.SemaphoreType.DMA((N_DEV - 1,)),
            pltpu.SemaphoreType.DMA((N_DEV - 1,)),
        ],
        compiler_params=pltpu.CompilerParams(collective_id=0),
    )(x2, wq_loc, k3, v3, wo_loc)
    return out2[None, :, :]


# device time: 73935 ns/iter; 1.9123x vs baseline; 1.9123x over previous
import jax
import jax.numpy as jnp
from jax import lax
from jax.experimental import pallas as pl
from jax.experimental.pallas import tpu as pltpu

N_DEV = 8
SQ = 1024
SKV = 1024
HQ_LOCAL = 8
DH = 128
D_MODEL = 1024
D_FF_LOCAL = HQ_LOCAL * DH
SCALE = 0.08838834764831843

PART_ROWS = ((0, 384), (384, 384), (768, 256))
PART_ORDER = (("x", "y", "z"), ("y", "z", "x"), ("z", "x", "y"))

_sem_signal = getattr(pl, "semaphore_signal", None) or pltpu.semaphore_signal
_sem_wait = getattr(pl, "semaphore_wait", None) or pltpu.semaphore_wait


def _stg_off(rows: int, r: int) -> int:
    return (0, rows >> 1, (rows >> 1) + (rows >> 2))[r]


def kernel(x, Wq, K_ext, V_ext, Wo):
    x2 = x[0]
    k3 = K_ext[0]
    v3 = V_ext[0]

    def body(x_ref, wq_ref, k_ref, v_ref, wo_ref, out_ref,
             acc_ref, stg0, stg1, stg2, rs_send, rs_recv, ag_send, ag_recv):
        stg = (stg0, stg1, stg2)
        my = lax.axis_index("i")

        m4 = my % 4
        bx = jnp.where((m4 == 1) | (m4 == 2), 1, 0)
        by = m4 // 2
        bz = my // 4
        partners = {
            "x": my + 1 - 2 * (my % 2),
            "y": my + 3 - 2 * m4,
            "z": my + 4 - 8 * bz,
        }
        bits = {"x": bx, "y": by, "z": bz}

        barrier = pltpu.get_barrier_semaphore()
        for dim in ("x", "y", "z"):
            _sem_signal(barrier, inc=1, device_id=(partners[dim],),
                        device_id_type=pl.DeviceIdType.MESH)
        _sem_wait(barrier, 3)

        q = jnp.dot(x_ref[:, :], wq_ref[:, :],
                    preferred_element_type=jnp.float32)

        qi = lax.broadcasted_iota(jnp.int32, (SQ, SKV), 0)
        ki = lax.broadcasted_iota(jnp.int32, (SQ, SKV), 1)
        mask = (jnp.abs(qi - ki) <= 128) | (ki < 32) | (qi < 32)

        ctx_heads = []
        for h in range(HQ_LOCAL):
            q_h = q[:, h * DH:(h + 1) * DH]
            k_h = k_ref[:, h, :]
            v_h = v_ref[:, h, :]
            s = jax.lax.dot_general(
                q_h, k_h, (((1,), (1,)), ((), ())),
                preferred_element_type=jnp.float32) * SCALE
            e = jnp.where(mask, jnp.exp(s), 0.0)
            denom = jnp.sum(e, axis=-1, keepdims=True)
            ctx_h = jnp.dot(e, v_h, preferred_element_type=jnp.float32) / denom
            ctx_heads.append(ctx_h)
        ctx = jnp.concatenate(ctx_heads, axis=1)

        def rs_rdma(j, r, src_off):
            rows = PART_ROWS[j][1]
            half = rows >> (r + 1)
            dim = PART_ORDER[j][r]
            return pltpu.make_async_remote_copy(
                src_ref=acc_ref.at[pl.ds(src_off, half), :],
                dst_ref=stg[j].at[pl.ds(_stg_off(rows, r), half), :],
                send_sem=rs_send.at[j, r],
                recv_sem=rs_recv.at[j, r],
                device_id=(partners[dim],),
                device_id_type=pl.DeviceIdType.MESH,
            )

        base = [off for off, _ in PART_ROWS]
        pend = [None, None, None]
        for j, (off0, rows) in enumerate(PART_ROWS):
            acc_ref[pl.ds(off0, rows), :] = jnp.dot(
                ctx[off0:off0 + rows, :], wo_ref[:, :],
                preferred_element_type=jnp.float32)
            b = bits[PART_ORDER[j][0]]
            rd = rs_rdma(j, 0, off0 + (1 - b) * (rows >> 1))
            rd.start()
            pend[j] = rd
        for r in range(3):
            nxt = [None, None, None]
            for j, (off0, rows) in enumerate(PART_ROWS):
                half = rows >> (r + 1)
                b = bits[PART_ORDER[j][r]]
                keep_off = base[j] + b * half
                pend[j].wait_recv()
                acc_ref[pl.ds(keep_off, half), :] = (
                    acc_ref[pl.ds(keep_off, half), :]
                    + stg[j][pl.ds(_stg_off(rows, r), half), :]
                )
                base[j] = keep_off
                if r < 2:
                    nb = bits[PART_ORDER[j][r + 1]]
                    nhalf = rows >> (r + 2)
                    rd = rs_rdma(j, r + 1, keep_off + (1 - nb) * nhalf)
                    rd.start()
                    nxt[j] = rd
            for j in range(3):
                pend[j].wait_send()
            pend = nxt

        for j, (off0, rows) in enumerate(PART_ROWS):
            blk = rows >> 3
            out_ref[pl.ds(base[j], blk), :] = acc_ref[pl.ds(base[j], blk), :]
            dim = PART_ORDER[j][2]
            rd = pltpu.make_async_remote_copy(
                src_ref=out_ref.at[pl.ds(base[j], blk), :],
                dst_ref=out_ref.at[pl.ds(base[j], blk), :],
                send_sem=ag_send.at[j, 2],
                recv_sem=ag_recv.at[j, 2],
                device_id=(partners[dim],),
                device_id_type=pl.DeviceIdType.MESH,
            )
            rd.start()
            pend[j] = rd

        for r in (2, 1, 0):
            nxt = [None, None, None]
            for j, (off0, rows) in enumerate(PART_ROWS):
                half = rows >> (r + 1)
                b = bits[PART_ORDER[j][r]]
                pend[j].wait_recv()
                base[j] = base[j] - b * half
                if r > 0:
                    ndim = PART_ORDER[j][r - 1]
                    rd = pltpu.make_async_remote_copy(
                        src_ref=out_ref.at[pl.ds(base[j], 2 * half), :],
                        dst_ref=out_ref.at[pl.ds(base[j], 2 * half), :],
                        send_sem=ag_send.at[j, r - 1],
                        recv_sem=ag_recv.at[j, r - 1],
                        device_id=(partners[ndim],),
                        device_id_type=pl.DeviceIdType.MESH,
                    )
                    rd.start()
                    nxt[j] = rd
            for j in range(3):
                pend[j].wait_send()
            pend = nxt

    out2 = pl.pallas_call(
        body,
        out_shape=jax.ShapeDtypeStruct((SQ, D_MODEL), jnp.float32),
        grid=(1,),
        in_specs=[
            pl.BlockSpec((SQ, D_MODEL), lambda i: (0, 0)),
            pl.BlockSpec((SQ, D_FF_LOCAL),
                         lambda i: (0, lax.axis_index("i"))),
            pl.BlockSpec((SKV, HQ_LOCAL, DH), lambda i: (0, 0, 0)),
            pl.BlockSpec((SKV, HQ_LOCAL, DH), lambda i: (0, 0, 0)),
            pl.BlockSpec((D_FF_LOCAL, D_MODEL),
                         lambda i: (lax.axis_index("i"), 0)),
        ],
        out_specs=pl.BlockSpec((SQ, D_MODEL), lambda i: (0, 0)),
        scratch_shapes=[
            pltpu.VMEM((SQ, D_MODEL), jnp.float32),
            pltpu.VMEM((336, D_MODEL), jnp.float32),
            pltpu.VMEM((336, D_MODEL), jnp.float32),
            pltpu.VMEM((224, D_MODEL), jnp.float32),
            pltpu.SemaphoreType.DMA((3, 3)),
            pltpu.SemaphoreType.DMA((3, 3)),
            pltpu.SemaphoreType.DMA((3, 3)),
            pltpu.SemaphoreType.DMA((3, 3)),
        ],
        compiler_params=pltpu.CompilerParams(collective_id=0),
    )(x2, Wq, k3, v3, Wo)
    return out2[None, :, :]
